# baseline (speedup 1.0000x reference)
import functools

import jax
import jax.numpy as jnp
from jax.experimental import pallas as pl

_N = 4096
_F = 256
_C = 40
_NB1 = 25
_NB2 = 10
_BM = 256
_CK = 256


def _lrelu(x):
    return jnp.where(x >= 0, x, 0.01 * x)


def _dot(a, b):
    return jax.lax.dot_general(a, b, (((1,), (0,)), ((), ())),
                               preferred_element_type=jnp.float32)


def _split(x):
    hi = x.astype(jnp.bfloat16)
    lo = (x - hi.astype(jnp.float32)).astype(jnp.bfloat16)
    return hi, lo


def _dot3(x, whi, wlo):
    xhi, xlo = _split(x)
    return _dot(xhi, whi) + (_dot(xhi, wlo) + _dot(xlo, whi))


def _layer1_body(a_ref, xhi_ref, xlo_ref, xbhi_ref, xblo_ref,
                 wnThi_ref, wnTlo_ref, bn_ref, wThi_ref, wTlo_ref, b_ref,
                 h_ref, sel2_ref, cnt_ref):
    r = jax.lax.broadcasted_iota(jnp.int32, (_CK, _CK), 0)
    c = jax.lax.broadcasted_iota(jnp.int32, (_CK, _CK), 1)
    tri = (r <= c).astype(jnp.bfloat16)
    carry = jnp.zeros((_BM, 1), jnp.float32)
    sel_chunks = []
    for ci in range(_N // _CK):
        a_c = a_ref[:, ci * _CK:(ci + 1) * _CK]
        csum = (_dot(a_c, tri) + carry).astype(jnp.bfloat16)
        sel1 = jnp.where(csum <= jnp.bfloat16(_NB1), a_c, jnp.bfloat16(0))
        sel2 = jnp.where(csum <= jnp.bfloat16(_NB2), a_c, jnp.bfloat16(0))
        sel2_ref[:, ci * _CK:(ci + 1) * _CK] = sel2.astype(jnp.int8)
        sel_chunks.append(sel1)
        carry = carry + jnp.sum(a_c, axis=1, keepdims=True).astype(jnp.float32)
    sel = jnp.concatenate(sel_chunks, axis=1)
    acc = _dot(sel, xhi_ref[...]) + _dot(sel, xlo_ref[...])
    cnt = jnp.minimum(carry, float(_NB1))
    mean = acc / jnp.maximum(cnt, 1.0)
    xj = _lrelu(_dot3(mean, wnThi_ref[...], wnTlo_ref[...]) + bn_ref[...])
    xi = _lrelu(_dot(xbhi_ref[...], wThi_ref[...])
                + (_dot(xbhi_ref[...], wTlo_ref[...])
                   + _dot(xblo_ref[...], wThi_ref[...])) + b_ref[...])
    h_ref[...] = xi + jnp.where(carry > 0, xj, 0.0)
    cnt_ref[...] = carry


def _layer2_body(sel2_ref, hhi_ref, hlo_ref, hbhi_ref, hblo_ref, cnt_ref,
                 wnThi_ref, wnTlo_ref, bn_ref, wThi_ref, wTlo_ref, b_ref,
                 w3Thi_ref, w3Tlo_ref, b3_ref, o_ref):
    total = cnt_ref[...]
    sel = sel2_ref[...].astype(jnp.bfloat16)
    acc = _dot(sel, hhi_ref[...]) + _dot(sel, hlo_ref[...])
    cnt = jnp.minimum(total, float(_NB2))
    mean = acc / jnp.maximum(cnt, 1.0)
    xj = _lrelu(_dot3(mean, wnThi_ref[...], wnTlo_ref[...]) + bn_ref[...])
    xi = _lrelu(_dot(hbhi_ref[...], wThi_ref[...])
                + (_dot(hbhi_ref[...], wTlo_ref[...])
                   + _dot(hblo_ref[...], wThi_ref[...])) + b_ref[...])
    h2 = xi + jnp.where(total > 0, xj, 0.0)
    logits = _dot3(h2, w3Thi_ref[...], w3Tlo_ref[...]) + b3_ref[...]
    m = jnp.max(logits, axis=1, keepdims=True)
    shifted = logits - m
    lse = jnp.log(jnp.sum(jnp.exp(shifted), axis=1, keepdims=True))
    o_ref[...] = shifted - lse


def _full(shape):
    return pl.BlockSpec(shape, lambda i: (0, 0))


def kernel(X, A, Wn1, bn1, W1, b1, Wn2, bn2, W2, b2, W3, b3):
    grid = (_N // _BM,)
    row_block = lambda i: (i, 0)
    Xhi, Xlo = _split(X)
    A_bf = A.astype(jnp.bfloat16)
    Wn1Thi, Wn1Tlo = _split(Wn1.T)
    W1Thi, W1Tlo = _split(W1.T)
    Wn2Thi, Wn2Tlo = _split(Wn2.T)
    W2Thi, W2Tlo = _split(W2.T)
    W3Thi, W3Tlo = _split(W3.T)

    h, sel2, cnt = pl.pallas_call(
        _layer1_body,
        grid=grid,
        in_specs=[
            pl.BlockSpec((_BM, _N), row_block),
            _full((_N, _F)),
            _full((_N, _F)),
            pl.BlockSpec((_BM, _F), row_block),
            pl.BlockSpec((_BM, _F), row_block),
            _full((_F, _F)),
            _full((_F, _F)),
            _full((1, _F)),
            _full((_F, _F)),
            _full((_F, _F)),
            _full((1, _F)),
        ],
        out_specs=[
            pl.BlockSpec((_BM, _F), row_block),
            pl.BlockSpec((_BM, _N), row_block),
            pl.BlockSpec((_BM, 1), row_block),
        ],
        out_shape=[
            jax.ShapeDtypeStruct((_N, _F), jnp.float32),
            jax.ShapeDtypeStruct((_N, _N), jnp.int8),
            jax.ShapeDtypeStruct((_N, 1), jnp.float32),
        ],
    )(A_bf, Xhi, Xlo, Xhi, Xlo, Wn1Thi, Wn1Tlo, bn1[None, :],
      W1Thi, W1Tlo, b1[None, :])

    hhi, hlo = _split(h)
    out = pl.pallas_call(
        _layer2_body,
        grid=grid,
        in_specs=[
            pl.BlockSpec((_BM, _N), row_block),
            _full((_N, _F)),
            _full((_N, _F)),
            pl.BlockSpec((_BM, _F), row_block),
            pl.BlockSpec((_BM, _F), row_block),
            pl.BlockSpec((_BM, 1), row_block),
            _full((_F, _F)),
            _full((_F, _F)),
            _full((1, _F)),
            _full((_F, _F)),
            _full((_F, _F)),
            _full((1, _F)),
            _full((_F, _C)),
            _full((_F, _C)),
            _full((1, _C)),
        ],
        out_specs=pl.BlockSpec((_BM, _C), row_block),
        out_shape=jax.ShapeDtypeStruct((_N, _C), jnp.float32),
    )(sel2, hhi, hlo, hhi, hlo, cnt, Wn2Thi, Wn2Tlo, bn2[None, :],
      W2Thi, W2Tlo, b2[None, :], W3Thi, W3Tlo, b3[None, :])
    return out


# speedup vs baseline: 2.9943x; 2.9943x over previous
import functools

import jax
import jax.numpy as jnp
from jax.experimental import pallas as pl

_N = 4096
_F = 256
_C = 40
_NB1 = 25
_NB2 = 10
_BM = 256
_CK = 256
_WFAST = 256


def _lrelu(x):
    return jnp.where(x >= 0, x, 0.01 * x)


def _dot(a, b):
    return jax.lax.dot_general(a, b, (((1,), (0,)), ((), ())),
                               preferred_element_type=jnp.float32)


def _split(x):
    hi = x.astype(jnp.bfloat16)
    lo = (x - hi.astype(jnp.float32)).astype(jnp.bfloat16)
    return hi, lo


def _dot3(x, w):
    xhi, xlo = _split(x)
    whi, wlo = _split(w)
    return _dot(xhi, whi) + (_dot(xhi, wlo) + _dot(xlo, whi))


def _sel_agg(a_ref, xa_ref, width, nb, sel2_ref):
    r = jax.lax.broadcasted_iota(jnp.int32, (_CK, _CK), 0)
    c = jax.lax.broadcasted_iota(jnp.int32, (_CK, _CK), 1)
    tri = (r <= c).astype(jnp.bfloat16)
    carry = jnp.zeros((_BM, 1), jnp.float32)
    sel_chunks = []
    for ci in range(width // _CK):
        a_c = a_ref[:, ci * _CK:(ci + 1) * _CK].astype(jnp.bfloat16)
        csum = (_dot(a_c, tri) + carry).astype(jnp.bfloat16)
        sel1 = jnp.where(csum <= jnp.bfloat16(nb), a_c, jnp.bfloat16(0))
        sel2 = jnp.where(csum <= jnp.bfloat16(_NB2), a_c, jnp.bfloat16(0))
        sel2_ref[:, ci * _CK:(ci + 1) * _CK] = sel2.astype(jnp.int8)
        sel_chunks.append(sel1)
        carry = carry + jnp.sum(a_c, axis=1, keepdims=True).astype(jnp.float32)
    sel = sel_chunks[0] if len(sel_chunks) == 1 else jnp.concatenate(
        sel_chunks, axis=1)
    xa = xa_ref[...]
    xhi, xlo = _split(xa)
    acc = _dot(sel, xhi) + _dot(sel, xlo)
    return acc, carry


def _mk_layer1_body(width, dyn_cnt):
    def body(a_ref, xa_ref, xb_ref, wnT_ref, bn_ref, wT_ref, b_ref,
             h_ref, sel2_ref, cnt_ref):
        acc, carry = _sel_agg(a_ref, xa_ref, width, _NB1, sel2_ref)
        if dyn_cnt:
            cnt = jnp.minimum(carry, float(_NB1))
            mean = acc / jnp.maximum(cnt, 1.0)
        else:
            mean = acc * (1.0 / _NB1)
        xj = _lrelu(_dot3(mean, wnT_ref[...]) + bn_ref[...])
        xi = _lrelu(_dot3(xb_ref[...], wT_ref[...]) + b_ref[...])
        if dyn_cnt:
            h_ref[...] = xi + jnp.where(carry > 0, xj, 0.0)
        else:
            h_ref[...] = xi + xj
        cnt_ref[...] = carry
    return body


def _mk_layer2_body(width, dyn_cnt):
    def body(sel2_ref, ha_ref, hb_ref, cnt_ref, wnT_ref, bn_ref, wT_ref,
             b_ref, w3T_ref, b3_ref, o_ref):
        total = cnt_ref[...]
        sel = sel2_ref[...].astype(jnp.bfloat16)
        ha = ha_ref[...]
        hhi, hlo = _split(ha)
        acc = _dot(sel, hhi) + _dot(sel, hlo)
        if dyn_cnt:
            cnt = jnp.minimum(total, float(_NB2))
            mean = acc / jnp.maximum(cnt, 1.0)
        else:
            mean = acc * (1.0 / _NB2)
        xj = _lrelu(_dot3(mean, wnT_ref[...]) + bn_ref[...])
        xi = _lrelu(_dot3(hb_ref[...], wT_ref[...]) + b_ref[...])
        if dyn_cnt:
            h2 = xi + jnp.where(total > 0, xj, 0.0)
        else:
            h2 = xi + xj
        logits = _dot3(h2, w3T_ref[...]) + b3_ref[...]
        m = jnp.max(logits, axis=1, keepdims=True)
        shifted = logits - m
        lse = jnp.log(jnp.sum(jnp.exp(shifted), axis=1, keepdims=True))
        o_ref[...] = shifted - lse
    return body


def _full(shape):
    return pl.BlockSpec(shape, lambda i: (0, 0))


_ROW = lambda i: (i, 0)


def _run_layer1(width, dyn_cnt, A, X, Wn1T, bn1, W1T, b1):
    grid = (_N // _BM,)
    return pl.pallas_call(
        _mk_layer1_body(width, dyn_cnt),
        grid=grid,
        in_specs=[
            pl.BlockSpec((_BM, width), _ROW),
            _full((width, _F)),
            pl.BlockSpec((_BM, _F), _ROW),
            _full((_F, _F)),
            _full((1, _F)),
            _full((_F, _F)),
            _full((1, _F)),
        ],
        out_specs=[
            pl.BlockSpec((_BM, _F), _ROW),
            pl.BlockSpec((_BM, width), _ROW),
            pl.BlockSpec((_BM, 1), _ROW),
        ],
        out_shape=[
            jax.ShapeDtypeStruct((_N, _F), jnp.float32),
            jax.ShapeDtypeStruct((_N, width), jnp.int8),
            jax.ShapeDtypeStruct((_N, 1), jnp.float32),
        ],
    )(A, X, X, Wn1T, bn1, W1T, b1)


def _run_layer2(width, dyn_cnt, sel2, h, cnt, Wn2T, bn2, W2T, b2, W3T, b3):
    grid = (_N // _BM,)
    return pl.pallas_call(
        _mk_layer2_body(width, dyn_cnt),
        grid=grid,
        in_specs=[
            pl.BlockSpec((_BM, width), _ROW),
            _full((width, _F)),
            pl.BlockSpec((_BM, _F), _ROW),
            pl.BlockSpec((_BM, 1), _ROW),
            _full((_F, _F)),
            _full((1, _F)),
            _full((_F, _F)),
            _full((1, _F)),
            _full((_F, _C)),
            _full((1, _C)),
        ],
        out_specs=pl.BlockSpec((_BM, _C), _ROW),
        out_shape=jax.ShapeDtypeStruct((_N, _C), jnp.float32),
    )(sel2, h, h, cnt, Wn2T, bn2, W2T, b2, W3T, b3)


def kernel(X, A, Wn1, bn1, W1, b1, Wn2, bn2, W2, b2, W3, b3):
    Wn1T, W1T = Wn1.T, W1.T
    Wn2T, W2T = Wn2.T, W2.T
    W3T = W3.T
    bn1_, b1_ = bn1[None, :], b1[None, :]
    bn2_, b2_ = bn2[None, :], b2[None, :]
    b3_ = b3[None, :]

    h, sel2, cnt = _run_layer1(_WFAST, False, A, X, Wn1T, bn1_, W1T, b1_)
    ok = jnp.all(cnt >= float(_NB1))

    def fast_path(_):
        return _run_layer2(_WFAST, False, sel2, h, cnt, Wn2T, bn2_, W2T, b2_,
                           W3T, b3_)

    def slow_path(_):
        hs, sel2s, cnts = _run_layer1(_N, True, A, X, Wn1T, bn1_, W1T, b1_)
        return _run_layer2(_N, True, sel2s, hs, cnts, Wn2T, bn2_, W2T, b2_,
                           W3T, b3_)

    return jax.lax.cond(ok, fast_path, slow_path, None)


# speedup vs baseline: 3.8127x; 1.2733x over previous
import functools

import jax
import jax.numpy as jnp
from jax.experimental import pallas as pl

_N = 4096
_F = 256
_C = 40
_NB1 = 25
_NB2 = 10
_BM = 512
_CK = 256
_WFAST = 256


def _lrelu(x):
    return jnp.where(x >= 0, x, 0.01 * x)


def _dot(a, b):
    return jax.lax.dot_general(a, b, (((1,), (0,)), ((), ())),
                               preferred_element_type=jnp.float32)


def _split(x):
    hi = x.astype(jnp.bfloat16)
    lo = (x - hi.astype(jnp.float32)).astype(jnp.bfloat16)
    return hi, lo


def _dot3(x, w):
    xhi, xlo = _split(x)
    whi, wlo = _split(w)
    return _dot(xhi, whi) + (_dot(xhi, wlo) + _dot(xlo, whi))


def _sel_agg(a_ref, xa_ref, width, nb, sel2_ref):
    r = jax.lax.broadcasted_iota(jnp.int32, (_CK, _CK), 0)
    c = jax.lax.broadcasted_iota(jnp.int32, (_CK, _CK), 1)
    tri = (r <= c).astype(jnp.bfloat16)
    carry = jnp.zeros((_BM, 1), jnp.float32)
    sel_chunks = []
    for ci in range(width // _CK):
        a_c = a_ref[:, ci * _CK:(ci + 1) * _CK].astype(jnp.bfloat16)
        csum = (_dot(a_c, tri) + carry).astype(jnp.bfloat16)
        sel1 = jnp.where(csum <= jnp.bfloat16(nb), a_c, jnp.bfloat16(0))
        sel2 = jnp.where(csum <= jnp.bfloat16(_NB2), a_c, jnp.bfloat16(0))
        sel2_ref[:, ci * _CK:(ci + 1) * _CK] = sel2.astype(jnp.int8)
        sel_chunks.append(sel1)
        carry = carry + jnp.sum(a_c, axis=1, keepdims=True).astype(jnp.float32)
    sel = sel_chunks[0] if len(sel_chunks) == 1 else jnp.concatenate(
        sel_chunks, axis=1)
    xa = xa_ref[...]
    xhi, xlo = _split(xa)
    acc = _dot(sel, xhi) + _dot(sel, xlo)
    return acc, carry


def _mk_layer1_body(width, dyn_cnt):
    def body(a_ref, xa_ref, xb_ref, wnT_ref, bn_ref, wT_ref, b_ref,
             h_ref, sel2_ref, cnt_ref):
        acc, carry = _sel_agg(a_ref, xa_ref, width, _NB1, sel2_ref)
        if dyn_cnt:
            cnt = jnp.minimum(carry, float(_NB1))
            mean = acc / jnp.maximum(cnt, 1.0)
        else:
            mean = acc * (1.0 / _NB1)
        xj = _lrelu(_dot3(mean, wnT_ref[...]) + bn_ref[...])
        xi = _lrelu(_dot3(xb_ref[...], wT_ref[...]) + b_ref[...])
        if dyn_cnt:
            h_ref[...] = xi + jnp.where(carry > 0, xj, 0.0)
        else:
            h_ref[...] = xi + xj
        cnt_ref[...] = carry
    return body


def _mk_layer2_body(width, dyn_cnt):
    def body(sel2_ref, ha_ref, hb_ref, cnt_ref, wnT_ref, bn_ref, wT_ref,
             b_ref, w3T_ref, b3_ref, o_ref):
        total = cnt_ref[...]
        sel = sel2_ref[...].astype(jnp.bfloat16)
        ha = ha_ref[...]
        hhi, hlo = _split(ha)
        acc = _dot(sel, hhi) + _dot(sel, hlo)
        if dyn_cnt:
            cnt = jnp.minimum(total, float(_NB2))
            mean = acc / jnp.maximum(cnt, 1.0)
        else:
            mean = acc * (1.0 / _NB2)
        xj = _lrelu(_dot3(mean, wnT_ref[...]) + bn_ref[...])
        xi = _lrelu(_dot3(hb_ref[...], wT_ref[...]) + b_ref[...])
        if dyn_cnt:
            h2 = xi + jnp.where(total > 0, xj, 0.0)
        else:
            h2 = xi + xj
        logits = _dot3(h2, w3T_ref[...]) + b3_ref[...]
        m = jnp.max(logits, axis=1, keepdims=True)
        shifted = logits - m
        lse = jnp.log(jnp.sum(jnp.exp(shifted), axis=1, keepdims=True))
        o_ref[...] = shifted - lse
    return body


def _full(shape):
    return pl.BlockSpec(shape, lambda i: (0, 0))


_ROW = lambda i: (i, 0)


def _run_layer1(width, dyn_cnt, A, X, Wn1T, bn1, W1T, b1):
    grid = (_N // _BM,)
    return pl.pallas_call(
        _mk_layer1_body(width, dyn_cnt),
        grid=grid,
        in_specs=[
            pl.BlockSpec((_BM, width), _ROW),
            _full((width, _F)),
            pl.BlockSpec((_BM, _F), _ROW),
            _full((_F, _F)),
            _full((1, _F)),
            _full((_F, _F)),
            _full((1, _F)),
        ],
        out_specs=[
            pl.BlockSpec((_BM, _F), _ROW),
            pl.BlockSpec((_BM, width), _ROW),
            pl.BlockSpec((_BM, 1), _ROW),
        ],
        out_shape=[
            jax.ShapeDtypeStruct((_N, _F), jnp.float32),
            jax.ShapeDtypeStruct((_N, width), jnp.int8),
            jax.ShapeDtypeStruct((_N, 1), jnp.float32),
        ],
    )(A, X, X, Wn1T, bn1, W1T, b1)


def _run_layer2(width, dyn_cnt, sel2, h, cnt, Wn2T, bn2, W2T, b2, W3T, b3):
    grid = (_N // _BM,)
    return pl.pallas_call(
        _mk_layer2_body(width, dyn_cnt),
        grid=grid,
        in_specs=[
            pl.BlockSpec((_BM, width), _ROW),
            _full((width, _F)),
            pl.BlockSpec((_BM, _F), _ROW),
            pl.BlockSpec((_BM, 1), _ROW),
            _full((_F, _F)),
            _full((1, _F)),
            _full((_F, _F)),
            _full((1, _F)),
            _full((_F, _C)),
            _full((1, _C)),
        ],
        out_specs=pl.BlockSpec((_BM, _C), _ROW),
        out_shape=jax.ShapeDtypeStruct((_N, _C), jnp.float32),
    )(sel2, h, h, cnt, Wn2T, bn2, W2T, b2, W3T, b3)


def kernel(X, A, Wn1, bn1, W1, b1, Wn2, bn2, W2, b2, W3, b3):
    Wn1T, W1T = Wn1.T, W1.T
    Wn2T, W2T = Wn2.T, W2.T
    W3T = W3.T
    bn1_, b1_ = bn1[None, :], b1[None, :]
    bn2_, b2_ = bn2[None, :], b2[None, :]
    b3_ = b3[None, :]

    h, sel2, cnt = _run_layer1(_WFAST, False, A, X, Wn1T, bn1_, W1T, b1_)
    ok = jnp.all(cnt >= float(_NB1))

    def fast_path(_):
        return _run_layer2(_WFAST, False, sel2, h, cnt, Wn2T, bn2_, W2T, b2_,
                           W3T, b3_)

    def slow_path(_):
        hs, sel2s, cnts = _run_layer1(_N, True, A, X, Wn1T, bn1_, W1T, b1_)
        return _run_layer2(_N, True, sel2s, hs, cnts, Wn2T, bn2_, W2T, b2_,
                           W3T, b3_)

    return jax.lax.cond(ok, fast_path, slow_path, None)


# speedup vs baseline: 4.2159x; 1.1058x over previous
import functools

import jax
import jax.numpy as jnp
from jax.experimental import pallas as pl

_N = 4096
_F = 256
_C = 40
_NB1 = 25
_NB2 = 10
_BMF = 1024
_BMS = 256
_CK = 256
_WFAST = 256


def _lrelu(x):
    return jnp.where(x >= 0, x, 0.01 * x)


def _dot(a, b):
    return jax.lax.dot_general(a, b, (((1,), (0,)), ((), ())),
                               preferred_element_type=jnp.float32)


def _split(x):
    hi = x.astype(jnp.bfloat16)
    lo = (x - hi.astype(jnp.float32)).astype(jnp.bfloat16)
    return hi, lo


def _dot3(x, w):
    xhi, xlo = _split(x)
    whi, wlo = _split(w)
    return _dot(xhi, whi) + (_dot(xhi, wlo) + _dot(xlo, whi))


def _sel_agg(a_ref, xa_ref, bm, width, nb, sel2_ref):
    r = jax.lax.broadcasted_iota(jnp.int32, (_CK, _CK), 0)
    c = jax.lax.broadcasted_iota(jnp.int32, (_CK, _CK), 1)
    tri = (r <= c).astype(jnp.bfloat16)
    carry = jnp.zeros((bm, 1), jnp.float32)
    sel_chunks = []
    for ci in range(width // _CK):
        a_c = a_ref[:, ci * _CK:(ci + 1) * _CK].astype(jnp.bfloat16)
        csum = (_dot(a_c, tri) + carry).astype(jnp.bfloat16)
        sel1 = jnp.where(csum <= jnp.bfloat16(nb), a_c, jnp.bfloat16(0))
        sel2 = jnp.where(csum <= jnp.bfloat16(_NB2), a_c, jnp.bfloat16(0))
        sel2_ref[:, ci * _CK:(ci + 1) * _CK] = sel2.astype(jnp.int8)
        sel_chunks.append(sel1)
        carry = carry + jnp.sum(a_c, axis=1, keepdims=True).astype(jnp.float32)
    sel = sel_chunks[0] if len(sel_chunks) == 1 else jnp.concatenate(
        sel_chunks, axis=1)
    xa = xa_ref[...]
    xhi, xlo = _split(xa)
    acc = _dot(sel, xhi) + _dot(sel, xlo)
    return acc, carry


def _mk_layer1_body(bm, width, dyn_cnt):
    def body(a_ref, xa_ref, xb_ref, wnT_ref, bn_ref, wT_ref, b_ref,
             h_ref, sel2_ref, cnt_ref):
        acc, carry = _sel_agg(a_ref, xa_ref, bm, width, _NB1, sel2_ref)
        if dyn_cnt:
            cnt = jnp.minimum(carry, float(_NB1))
            mean = acc / jnp.maximum(cnt, 1.0)
        else:
            mean = acc * (1.0 / _NB1)
        xj = _lrelu(_dot3(mean, wnT_ref[...]) + bn_ref[...])
        xi = _lrelu(_dot3(xb_ref[...], wT_ref[...]) + b_ref[...])
        if dyn_cnt:
            h_ref[...] = xi + jnp.where(carry > 0, xj, 0.0)
        else:
            h_ref[...] = xi + xj
        cnt_ref[...] = carry
    return body


def _mk_layer2_body(bm, width, dyn_cnt):
    def body(sel2_ref, ha_ref, hb_ref, cnt_ref, wnT_ref, bn_ref, wT_ref,
             b_ref, w3T_ref, b3_ref, o_ref):
        total = cnt_ref[...]
        sel = sel2_ref[...].astype(jnp.bfloat16)
        ha = ha_ref[...]
        hhi, hlo = _split(ha)
        acc = _dot(sel, hhi) + _dot(sel, hlo)
        if dyn_cnt:
            cnt = jnp.minimum(total, float(_NB2))
            mean = acc / jnp.maximum(cnt, 1.0)
        else:
            mean = acc * (1.0 / _NB2)
        xj = _lrelu(_dot3(mean, wnT_ref[...]) + bn_ref[...])
        xi = _lrelu(_dot3(hb_ref[...], wT_ref[...]) + b_ref[...])
        if dyn_cnt:
            h2 = xi + jnp.where(total > 0, xj, 0.0)
        else:
            h2 = xi + xj
        logits = _dot3(h2, w3T_ref[...]) + b3_ref[...]
        m = jnp.max(logits, axis=1, keepdims=True)
        shifted = logits - m
        lse = jnp.log(jnp.sum(jnp.exp(shifted), axis=1, keepdims=True))
        o_ref[...] = shifted - lse
    return body


def _full(shape):
    return pl.BlockSpec(shape, lambda i: (0, 0))


_ROW = lambda i: (i, 0)


def _run_layer1(bm, width, dyn_cnt, A, X, Wn1T, bn1, W1T, b1):
    grid = (_N // bm,)
    return pl.pallas_call(
        _mk_layer1_body(bm, width, dyn_cnt),
        grid=grid,
        in_specs=[
            pl.BlockSpec((bm, width), _ROW),
            _full((width, _F)),
            pl.BlockSpec((bm, _F), _ROW),
            _full((_F, _F)),
            _full((1, _F)),
            _full((_F, _F)),
            _full((1, _F)),
        ],
        out_specs=[
            pl.BlockSpec((bm, _F), _ROW),
            pl.BlockSpec((bm, width), _ROW),
            pl.BlockSpec((bm, 1), _ROW),
        ],
        out_shape=[
            jax.ShapeDtypeStruct((_N, _F), jnp.float32),
            jax.ShapeDtypeStruct((_N, width), jnp.int8),
            jax.ShapeDtypeStruct((_N, 1), jnp.float32),
        ],
    )(A, X, X, Wn1T, bn1, W1T, b1)


def _run_layer2(bm, width, dyn_cnt, sel2, h, cnt, Wn2T, bn2, W2T, b2, W3T, b3):
    grid = (_N // bm,)
    return pl.pallas_call(
        _mk_layer2_body(bm, width, dyn_cnt),
        grid=grid,
        in_specs=[
            pl.BlockSpec((bm, width), _ROW),
            _full((width, _F)),
            pl.BlockSpec((bm, _F), _ROW),
            pl.BlockSpec((bm, 1), _ROW),
            _full((_F, _F)),
            _full((1, _F)),
            _full((_F, _F)),
            _full((1, _F)),
            _full((_F, _C)),
            _full((1, _C)),
        ],
        out_specs=pl.BlockSpec((bm, _C), _ROW),
        out_shape=jax.ShapeDtypeStruct((_N, _C), jnp.float32),
    )(sel2, h, h, cnt, Wn2T, bn2, W2T, b2, W3T, b3)


def kernel(X, A, Wn1, bn1, W1, b1, Wn2, bn2, W2, b2, W3, b3):
    Wn1T, W1T = Wn1.T, W1.T
    Wn2T, W2T = Wn2.T, W2.T
    W3T = W3.T
    bn1_, b1_ = bn1[None, :], b1[None, :]
    bn2_, b2_ = bn2[None, :], b2[None, :]
    b3_ = b3[None, :]

    h, sel2, cnt = _run_layer1(_BMF, _WFAST, False, A, X, Wn1T, bn1_, W1T, b1_)
    ok = jnp.all(cnt >= float(_NB1))

    def fast_path(_):
        return _run_layer2(_BMF, _WFAST, False, sel2, h, cnt, Wn2T, bn2_, W2T,
                           b2_, W3T, b3_)

    def slow_path(_):
        hs, sel2s, cnts = _run_layer1(_BMS, _N, True, A, X, Wn1T, bn1_, W1T, b1_)
        return _run_layer2(_BMS, _N, True, sel2s, hs, cnts, Wn2T, bn2_, W2T,
                           b2_, W3T, b3_)

    return jax.lax.cond(ok, fast_path, slow_path, None)
